# baseline (speedup 1.0000x reference)
import jax
import jax.numpy as jnp
from jax import lax
from jax.experimental import pallas as pl
from jax.experimental.pallas import tpu as pltpu
from jax.experimental.pallas import tpu_sc as plsc

NUM_EMB = 1000001
EMB_DIM = 16
BATCH = 16384
SEQ = 200

NC = 2
NS = 16
NW = NC * NS
RPW = BATCH // NW
C_L = 25
N_CHUNKS = SEQ // C_L

CB = 65536
G1 = 16


def _project_body(t_ref, w_ref, out_ref):
    s = jnp.sum(t_ref[...] * w_ref[...], axis=0) * (1.0 / SEQ)
    out_ref[...] = s.astype(jnp.bfloat16)


def _project(tableT, W):
    return pl.pallas_call(
        _project_body,
        grid=(G1,),
        in_specs=[
            pl.BlockSpec((EMB_DIM, CB), lambda i: (0, i)),
            pl.BlockSpec((EMB_DIM, 1), lambda i: (0, 0)),
        ],
        out_specs=pl.BlockSpec((CB,), lambda i: (i,)),
        out_shape=jax.ShapeDtypeStruct((G1 * CB,), jnp.bfloat16),
    )(tableT, W)


C_IDX = C_L * RPW


NWORDS = (G1 * CB) // 2


def _pool_body(idx_hbm, tv_hbm, b_hbm, out_hbm,
               idx_v0, idx_v1, gidx_v0, gidx_v1, vals_v0, vals_v1,
               bias_v, acc_v, tv_sh,
               sem0, sem1, isem0, isem1):
    sid = lax.axis_index("s")
    w = sid * NC + lax.axis_index("c")
    col0 = w * RPW
    idx_bufs = (idx_v0, idx_v1)
    gidx_bufs = (gidx_v0, gidx_v1)
    val_bufs = (vals_v0, vals_v1)
    sem_bufs = (sem0, sem1)
    isem_bufs = (isem0, isem1)

    seg = NWORDS // NS
    pltpu.sync_copy(tv_hbm.at[pl.ds(sid * seg, seg)],
                    tv_sh.at[pl.ds(sid * seg, seg)])

    pltpu.sync_copy(b_hbm, bias_v)
    bias = bias_v[...]
    zero = jnp.zeros((16,), jnp.float32)

    def zbody(g, x):
        acc_v[pl.ds(16 * g, 16)] = zero
        return x

    lax.fori_loop(0, RPW // 16, zbody, 0)
    plsc.subcore_barrier()

    def issue_idx(c, buf):
        base = c * C_L * BATCH + col0

        def body(l, x):
            pltpu.async_copy(
                idx_hbm.at[pl.ds(base + l * BATCH, RPW)],
                idx_bufs[buf].at[pl.ds(l * RPW, RPW)],
                isem_bufs[buf])
            return x

        lax.fori_loop(0, C_L, body, 0)

    def drain_idx(buf):
        pltpu.make_async_copy(idx_hbm.at[pl.ds(0, C_IDX)], idx_bufs[buf],
                              isem_bufs[buf]).wait()

    def shift_idx(buf):
        ib, gb = idx_bufs[buf], gidx_bufs[buf]

        def body(k, x):
            p = 16 * k
            gb[pl.ds(p, 16)] = ib[pl.ds(p, 16)] >> 1
            return x

        lax.fori_loop(0, C_IDX // 16, body, 0)

    def gather(buf):
        return pltpu.async_copy(tv_sh.at[gidx_bufs[buf]], val_bufs[buf],
                                sem_bufs[buf])

    issue_idx(0, 0)
    drain_idx(0)
    shift_idx(0)
    gdescs = {0: gather(0)}
    for c in range(N_CHUNKS):
        buf = c % 2
        if c + 1 < N_CHUNKS:
            nbuf = (c + 1) % 2
            issue_idx(c + 1, nbuf)
            drain_idx(nbuf)
            shift_idx(nbuf)
            gdescs[c + 1] = gather(nbuf)
        gdescs[c].wait()
        vb = val_bufs[buf]
        ib = idx_bufs[buf]
        one = jnp.full((16,), 1, jnp.int32)
        himask = jnp.full((16,), -65536, jnp.int32)

        def gbody(g, x):
            off = 16 * g

            def body(j, a):
                p = j * 5 * RPW + off
                for r in range(5):
                    q = p + r * RPW
                    v = vb[pl.ds(q, 16)]
                    parity = ib[pl.ds(q, 16)] & one
                    bits = jnp.where(parity == 0, v << 16, v & himask)
                    a = a + plsc.bitcast(bits, jnp.float32)
                return a

            part = lax.fori_loop(0, C_L // 5, body, zero)
            acc_v[pl.ds(off, 16)] += part
            return x

        lax.fori_loop(0, RPW // 16, gbody, 0)

    def bbody(g, x):
        acc_v[pl.ds(16 * g, 16)] += bias
        return x

    lax.fori_loop(0, RPW // 16, bbody, 0)
    pltpu.sync_copy(acc_v, out_hbm.at[pl.ds(col0, RPW)])


def _pool(idxTf, tv_flat, b16):
    mesh = plsc.VectorSubcoreMesh(core_axis_name="c", subcore_axis_name="s")
    f = pl.kernel(
        _pool_body,
        out_type=jax.ShapeDtypeStruct((BATCH,), jnp.float32),
        mesh=mesh,
        scratch_types=[
            pltpu.VMEM((C_IDX,), jnp.int32),
            pltpu.VMEM((C_IDX,), jnp.int32),
            pltpu.VMEM((C_IDX,), jnp.int32),
            pltpu.VMEM((C_IDX,), jnp.int32),
            pltpu.VMEM((C_IDX,), jnp.int32),
            pltpu.VMEM((C_IDX,), jnp.int32),
            pltpu.VMEM((16,), jnp.float32),
            pltpu.VMEM((RPW,), jnp.float32),
            pltpu.VMEM_SHARED((NWORDS,), jnp.int32),
            pltpu.SemaphoreType.DMA,
            pltpu.SemaphoreType.DMA,
            pltpu.SemaphoreType.DMA,
            pltpu.SemaphoreType.DMA,
        ],
        compiler_params=pltpu.CompilerParams(needs_layout_passes=False),
    )
    return f(idxTf, tv_flat, b16)


def kernel(indices, table, W, b):
    tableT = table.T
    tv_bf16 = _project(tableT, W.astype(jnp.float32))
    tv32 = lax.bitcast_convert_type(tv_bf16.reshape(NWORDS, 2),
                                    jnp.int32)
    idxTf = indices.astype(jnp.int32).T.reshape(SEQ * BATCH)
    b16 = jnp.broadcast_to(b.astype(jnp.float32), (16,))
    out = _pool(idxTf, tv32, b16)
    return out.reshape(BATCH, 1)


# speedup vs baseline: 2.4826x; 2.4826x over previous
import jax
import jax.numpy as jnp
from jax import lax
from jax.experimental import pallas as pl
from jax.experimental.pallas import tpu as pltpu
from jax.experimental.pallas import tpu_sc as plsc

NUM_EMB = 1000001
EMB_DIM = 16
BATCH = 16384
SEQ = 200

NC = 2
NS = 16
NW = NC * NS
RPW = BATCH // NW
C_L = 40
N_CHUNKS = SEQ // C_L

CB = 65536
G1 = 16


def _project_body(t_ref, w_ref, out_ref):
    out_ref[...] = jnp.sum(t_ref[...] * w_ref[...], axis=0) * (1.0 / SEQ)


def _project(tableT, W):
    return pl.pallas_call(
        _project_body,
        grid=(G1,),
        in_specs=[
            pl.BlockSpec((EMB_DIM, CB), lambda i: (0, i)),
            pl.BlockSpec((EMB_DIM, 1), lambda i: (0, 0)),
        ],
        out_specs=pl.BlockSpec((CB,), lambda i: (i,)),
        out_shape=jax.ShapeDtypeStruct((G1 * CB,), jnp.float32),
    )(tableT, W)


C_IDX = C_L * RPW


def _pool_body(idx_hbm, tv_hbm, b_hbm, out_hbm,
               idx_v0, idx_v1, vals_v0, vals_v1, bias_v, acc_v,
               sem0, sem1, isem0, isem1):
    sid = lax.axis_index("s")
    w = sid * NC + lax.axis_index("c")
    col0 = w * RPW
    idx_bufs = (idx_v0, idx_v1)
    val_bufs = (vals_v0, vals_v1)
    sem_bufs = (sem0, sem1)
    isem_bufs = (isem0, isem1)

    pltpu.sync_copy(b_hbm, bias_v)
    bias = bias_v[...]
    zero = jnp.zeros((16,), jnp.float32)

    def zbody(g, x):
        acc_v[pl.ds(16 * g, 16)] = zero
        return x

    lax.fori_loop(0, RPW // 16, zbody, 0)

    def issue_idx(c, buf):
        base = c * C_L * BATCH + col0

        def body(l, x):
            pltpu.async_copy(
                idx_hbm.at[pl.ds(base + l * BATCH, RPW)],
                idx_bufs[buf].at[pl.ds(l * RPW, RPW)],
                isem_bufs[buf])
            return x

        lax.fori_loop(0, C_L, body, 0)

    def drain_idx(buf):
        pltpu.make_async_copy(idx_hbm.at[pl.ds(0, C_IDX)], idx_bufs[buf],
                              isem_bufs[buf]).wait()

    def gather(buf):
        return pltpu.async_copy(tv_hbm.at[idx_bufs[buf]], val_bufs[buf],
                                sem_bufs[buf])

    issue_idx(0, 0)
    drain_idx(0)
    gdescs = {0: gather(0)}
    for c in range(N_CHUNKS):
        buf = c % 2
        if c + 1 < N_CHUNKS:
            nbuf = (c + 1) % 2
            issue_idx(c + 1, nbuf)
            drain_idx(nbuf)
            gdescs[c + 1] = gather(nbuf)
        gdescs[c].wait()
        vb = val_bufs[buf]

        def gbody(g, x):
            off = 16 * g

            def body(j, a):
                p = j * 5 * RPW + off
                for r in range(5):
                    a = a + vb[pl.ds(p + r * RPW, 16)]
                return a

            part = lax.fori_loop(0, C_L // 5, body, zero)
            acc_v[pl.ds(off, 16)] += part
            return x

        lax.fori_loop(0, RPW // 16, gbody, 0)

    def bbody(g, x):
        acc_v[pl.ds(16 * g, 16)] += bias
        return x

    lax.fori_loop(0, RPW // 16, bbody, 0)
    pltpu.sync_copy(acc_v, out_hbm.at[pl.ds(col0, RPW)])


def _pool(idxTf, tv_flat, b16):
    mesh = plsc.VectorSubcoreMesh(core_axis_name="c", subcore_axis_name="s")
    f = pl.kernel(
        _pool_body,
        out_type=jax.ShapeDtypeStruct((BATCH,), jnp.float32),
        mesh=mesh,
        scratch_types=[
            pltpu.VMEM((C_IDX,), jnp.int32),
            pltpu.VMEM((C_IDX,), jnp.int32),
            pltpu.VMEM((C_IDX,), jnp.float32),
            pltpu.VMEM((C_IDX,), jnp.float32),
            pltpu.VMEM((16,), jnp.float32),
            pltpu.VMEM((RPW,), jnp.float32),
            pltpu.SemaphoreType.DMA,
            pltpu.SemaphoreType.DMA,
            pltpu.SemaphoreType.DMA,
            pltpu.SemaphoreType.DMA,
        ],
        compiler_params=pltpu.CompilerParams(needs_layout_passes=False),
    )
    return f(idxTf, tv_flat, b16)


def kernel(indices, table, W, b):
    tableT = table.T
    tv_flat = _project(tableT, W.astype(jnp.float32))
    idxTf = indices.astype(jnp.int32).T.reshape(SEQ * BATCH)
    b16 = jnp.broadcast_to(b.astype(jnp.float32), (16,))
    out = _pool(idxTf, tv_flat, b16)
    return out.reshape(BATCH, 1)
